# baseline (speedup 1.0000x reference)
import jax
import jax.numpy as jnp
from jax.experimental import pallas as pl
from jax.experimental.pallas import tpu as pltpu

B = 128
N = 32768
BLOCK_ROWS = 32
GRID = B // BLOCK_ROWS


def _softmax_block(x_ref, o_hbm, scratch, sems):
    i = pl.program_id(0)
    slot = jax.lax.rem(i, 2)

    @pl.when(i >= 2)
    def _wait_prev():
        pltpu.make_async_copy(
            scratch.at[slot],
            o_hbm.at[pl.ds((i - 2) * BLOCK_ROWS, BLOCK_ROWS), :],
            sems.at[slot],
        ).wait()

    x = x_ref[...]
    m = jnp.max(x, axis=1, keepdims=True)
    e = jnp.exp(x - m)
    s = jnp.sum(e, axis=1, keepdims=True)
    scratch[slot, :, :N] = e * (1.0 / s)
    scratch[slot, :, N:] = jnp.zeros((BLOCK_ROWS, 1), jnp.float32)

    cp = pltpu.make_async_copy(
        scratch.at[slot],
        o_hbm.at[pl.ds(i * BLOCK_ROWS, BLOCK_ROWS), :],
        sems.at[slot],
    )
    cp.start()

    @pl.when(i == GRID - 1)
    def _drain():
        pltpu.make_async_copy(
            scratch.at[jax.lax.rem(i - 1, 2)],
            o_hbm.at[pl.ds((i - 1) * BLOCK_ROWS, BLOCK_ROWS), :],
            sems.at[jax.lax.rem(i - 1, 2)],
        ).wait()
        cp.wait()


def kernel(output_true):
    return pl.pallas_call(
        _softmax_block,
        grid=(GRID,),
        in_specs=[pl.BlockSpec((BLOCK_ROWS, N), lambda i: (i, 0))],
        out_specs=pl.BlockSpec(memory_space=pl.ANY),
        out_shape=jax.ShapeDtypeStruct((B, N + 1), output_true.dtype),
        scratch_shapes=[
            pltpu.VMEM((2, BLOCK_ROWS, N + 1), jnp.float32),
            pltpu.SemaphoreType.DMA((2,)),
        ],
    )(output_true)
